# Initial kernel scaffold; baseline (speedup 1.0000x reference)
import functools

import jax
import jax.numpy as jnp
from jax import lax
from jax.experimental import pallas as pl
from jax.experimental.pallas import tpu as pltpu
from jax.experimental.pallas import tpu_sc as plsc

N = 50000
E = 800000
D_IN = 128
H = 64
HH = 32
NB = 8

BLK = 2000
NP = 52000
GRID = NP // BLK
GRID_N = N // BLK
SINK = N

NC, NS = 2, 16
CHUNK = 128
E_PAD = 802816
AGG_STEPS = E_PAD // NS // CHUNK
DEG_STEPS = E_PAD // (NC * NS) // CHUNK
ROWS_PT = NP // NS

_f32 = jnp.float32


def _sc_mesh():
    return plsc.VectorSubcoreMesh(
        core_axis_name="c", subcore_axis_name="s", num_cores=NC, num_subcores=NS
    )


@functools.partial(
    pl.kernel,
    out_type=jax.ShapeDtypeStruct((NC, NP, 16), _f32),
    mesh=_sc_mesh(),
    scratch_types=[
        pltpu.VMEM((DEG_STEPS, CHUNK), jnp.int32),
        pltpu.VMEM((CHUNK, 16), _f32),
        pltpu.VMEM_SHARED((NP, 16), _f32),
    ],
)
def _sc_deg(dst3, ones_hbm, zeros_hbm, deg_out, dst_v, ones_v, acc):
    c = lax.axis_index("c")
    s = lax.axis_index("s")
    w = c * NS + s
    pltpu.sync_copy(dst3.at[w], dst_v)
    pltpu.sync_copy(ones_hbm, ones_v)
    pltpu.sync_copy(zeros_hbm, acc.at[pl.ds(s * ROWS_PT, ROWS_PT)])
    plsc.subcore_barrier()

    def st(j, carry):
        pltpu.sync_copy(ones_v, acc.at[dst_v.at[j]], add=True)
        return carry

    lax.fori_loop(0, DEG_STEPS, st, None)
    plsc.subcore_barrier()
    pltpu.sync_copy(
        acc.at[pl.ds(s * ROWS_PT, ROWS_PT)],
        deg_out.at[c, pl.ds(s * ROWS_PT, ROWS_PT)],
    )


@functools.partial(
    pl.kernel,
    out_type=(
        jax.ShapeDtypeStruct((NP, HH), _f32),
        jax.ShapeDtypeStruct((NP, HH), _f32),
    ),
    mesh=_sc_mesh(),
    scratch_types=[
        pltpu.VMEM((AGG_STEPS, CHUNK), jnp.int32),
        pltpu.VMEM((AGG_STEPS, CHUNK), jnp.int32),
        pltpu.VMEM((CHUNK, HH), _f32),
        pltpu.VMEM((CHUNK, HH), _f32),
        pltpu.VMEM_SHARED((NP, HH), _f32),
        pltpu.SemaphoreType.DMA,
        pltpu.SemaphoreType.DMA,
    ],
)
def _sc_agg(hA, hB, src3, dst3, aggA, aggB, src_v, dst_v, rows0, rows1, acc,
            sem0, sem1):
    c = lax.axis_index("c")
    s = lax.axis_index("s")
    pltpu.sync_copy(src3.at[s], src_v)
    pltpu.sync_copy(dst3.at[s], dst_v)

    def run(h_hbm, out_hbm):
        pltpu.sync_copy(
            h_hbm.at[pl.ds(s * ROWS_PT, ROWS_PT)],
            acc.at[pl.ds(s * ROWS_PT, ROWS_PT)],
        )
        plsc.subcore_barrier()
        pltpu.async_copy(h_hbm.at[src_v.at[0]], rows0, sem0)

        def pair(jj, carry):
            j0 = jj * 2
            pltpu.async_copy(h_hbm.at[src_v.at[j0 + 1]], rows1, sem1)
            pltpu.make_async_copy(h_hbm.at[src_v.at[j0]], rows0, sem0).wait()
            pltpu.sync_copy(rows0, acc.at[dst_v.at[j0]], add=True)

            @pl.when(j0 + 2 < AGG_STEPS)
            def _():
                pltpu.async_copy(h_hbm.at[src_v.at[j0 + 2]], rows0, sem0)

            pltpu.make_async_copy(h_hbm.at[src_v.at[j0 + 1]], rows1, sem1).wait()
            pltpu.sync_copy(rows1, acc.at[dst_v.at[j0 + 1]], add=True)
            return carry

        lax.fori_loop(0, AGG_STEPS // 2, pair, None)
        plsc.subcore_barrier()
        pltpu.sync_copy(
            acc.at[pl.ds(s * ROWS_PT, ROWS_PT)],
            out_hbm.at[pl.ds(s * ROWS_PT, ROWS_PT)],
        )

    @pl.when(c == 0)
    def _():
        run(hA, aggA)

    @pl.when(c == 1)
    def _():
        run(hB, aggB)


def _tc0_body(nodes_ref, win_ref, bin_ref, w0_ref, d0_ref, d1_ref,
              hA_ref, hB_ref, dis_ref):
    x = jnp.dot(nodes_ref[...], win_ref[...], preferred_element_type=_f32)
    x = x + bin_ref[...]
    deg = d0_ref[...][:, :1] + d1_ref[...][:, :1] + 1.0
    dis = lax.rsqrt(deg)
    h = jnp.dot(x, w0_ref[...], preferred_element_type=_f32) * dis
    hA_ref[...] = h[:, :HH]
    hB_ref[...] = h[:, HH:]
    dis_ref[...] = dis


def _tc0(nodes, W_in, b_in, W0, d0, d1):
    return pl.pallas_call(
        _tc0_body,
        grid=(GRID,),
        in_specs=[
            pl.BlockSpec((BLK, D_IN), lambda i: (jnp.minimum(i, GRID_N - 1), 0)),
            pl.BlockSpec((D_IN, H), lambda i: (0, 0)),
            pl.BlockSpec((1, H), lambda i: (0, 0)),
            pl.BlockSpec((H, H), lambda i: (0, 0)),
            pl.BlockSpec((BLK, 16), lambda i: (i, 0)),
            pl.BlockSpec((BLK, 16), lambda i: (i, 0)),
        ],
        out_specs=[
            pl.BlockSpec((BLK, HH), lambda i: (i, 0)),
            pl.BlockSpec((BLK, HH), lambda i: (i, 0)),
            pl.BlockSpec((BLK, 1), lambda i: (i, 0)),
        ],
        out_shape=[
            jax.ShapeDtypeStruct((NP, HH), _f32),
            jax.ShapeDtypeStruct((NP, HH), _f32),
            jax.ShapeDtypeStruct((NP, 1), _f32),
        ],
    )(nodes, W_in, b_in, W0, d0, d1)


def _post(aggA, aggB, dis, b, g, lb, xres):
    out = jnp.concatenate([aggA, aggB], axis=1) * dis + b
    mu = jnp.mean(out, axis=-1, keepdims=True)
    var = jnp.mean((out - mu) ** 2, axis=-1, keepdims=True)
    out = (out - mu) * lax.rsqrt(var + 1e-5) * g + lb
    out = jnp.maximum(out, 0.0)
    if xres is not None:
        out = out + xres
    return out


def _epi_body(with_res, *refs):
    if with_res:
        (aggA_ref, aggB_ref, dis_ref, b_ref, g_ref, lb_ref, wn_ref, xres_ref,
         x_ref, hA_ref, hB_ref) = refs
        xres = xres_ref[...]
    else:
        (aggA_ref, aggB_ref, dis_ref, b_ref, g_ref, lb_ref, wn_ref,
         x_ref, hA_ref, hB_ref) = refs
        xres = None
    dis = dis_ref[...]
    out = _post(aggA_ref[...], aggB_ref[...], dis, b_ref[...], g_ref[...],
                lb_ref[...], xres)
    x_ref[...] = out
    h = jnp.dot(out, wn_ref[...], preferred_element_type=_f32) * dis
    hA_ref[...] = h[:, :HH]
    hB_ref[...] = h[:, HH:]


def _epi(with_res):
    body = functools.partial(_epi_body, with_res)
    n_par = pl.BlockSpec((1, H), lambda i: (0, 0))
    in_specs = [
        pl.BlockSpec((BLK, HH), lambda i: (i, 0)),
        pl.BlockSpec((BLK, HH), lambda i: (i, 0)),
        pl.BlockSpec((BLK, 1), lambda i: (i, 0)),
        n_par, n_par, n_par,
        pl.BlockSpec((H, H), lambda i: (0, 0)),
    ]
    if with_res:
        in_specs.append(pl.BlockSpec((BLK, H), lambda i: (i, 0)))
    return pl.pallas_call(
        body,
        grid=(GRID,),
        in_specs=in_specs,
        out_specs=[
            pl.BlockSpec((BLK, H), lambda i: (i, 0)),
            pl.BlockSpec((BLK, HH), lambda i: (i, 0)),
            pl.BlockSpec((BLK, HH), lambda i: (i, 0)),
        ],
        out_shape=[
            jax.ShapeDtypeStruct((NP, H), _f32),
            jax.ShapeDtypeStruct((NP, HH), _f32),
            jax.ShapeDtypeStruct((NP, HH), _f32),
        ],
    )


def _fin_body(aggA_ref, aggB_ref, dis_ref, b_ref, g_ref, lb_ref, xres_ref,
              bi_ref, node_ref, plan_ref):
    out = _post(aggA_ref[...], aggB_ref[...], dis_ref[...], b_ref[...],
                g_ref[...], lb_ref[...], xres_ref[...])
    node_ref[...] = out
    oh = (bi_ref[...] == lax.broadcasted_iota(jnp.int32, (BLK, NB), 1))
    part = lax.dot_general(oh.astype(_f32), out, (((0,), (0,)), ((), ())),
                           preferred_element_type=_f32)

    @pl.when(pl.program_id(0) == 0)
    def _():
        plan_ref[...] = jnp.zeros_like(plan_ref)

    plan_ref[...] += part


def _fin(aggA, aggB, dis, b, g, lb, xres, bi):
    n_par = pl.BlockSpec((1, H), lambda i: (0, 0))
    return pl.pallas_call(
        _fin_body,
        grid=(GRID_N,),
        in_specs=[
            pl.BlockSpec((BLK, HH), lambda i: (i, 0)),
            pl.BlockSpec((BLK, HH), lambda i: (i, 0)),
            pl.BlockSpec((BLK, 1), lambda i: (i, 0)),
            n_par, n_par, n_par,
            pl.BlockSpec((BLK, H), lambda i: (i, 0)),
            pl.BlockSpec((BLK, 1), lambda i: (i, 0)),
        ],
        out_specs=[
            pl.BlockSpec((BLK, H), lambda i: (i, 0)),
            pl.BlockSpec((NB, H), lambda i: (0, 0)),
        ],
        out_shape=[
            jax.ShapeDtypeStruct((N, H), _f32),
            jax.ShapeDtypeStruct((NB, H), _f32),
        ],
    )(aggA, aggB, dis, b, g, lb, xres, bi)


def kernel(nodes, edges, edge_types, batch_idx, W_in, b_in, emb, gcn_W, gcn_b,
           ln_g, ln_b):
    del edge_types, emb
    pad = jnp.full((E_PAD - E,), SINK, jnp.int32)
    src_flat = jnp.concatenate([edges[0], pad])
    dst_flat = jnp.concatenate([edges[1], pad])
    src3 = src_flat.reshape(NS, AGG_STEPS, CHUNK)
    dst3 = dst_flat.reshape(NS, AGG_STEPS, CHUNK)
    dst_deg = dst_flat.reshape(NC * NS, DEG_STEPS, CHUNK)
    ones16 = jnp.ones((CHUNK, 16), _f32)
    zeros16 = jnp.zeros((ROWS_PT, 16), _f32)

    deg = _sc_deg(dst_deg, ones16, zeros16)
    hA, hB, dis = _tc0(nodes, W_in, b_in.reshape(1, H), gcn_W[0],
                       deg[0], deg[1])
    x = None
    node_emb = plan_emb = None
    for i in range(3):
        aggA, aggB = _sc_agg(hA, hB, src3, dst3)
        par = (dis, gcn_b[i].reshape(1, H), ln_g[i].reshape(1, H),
               ln_b[i].reshape(1, H))
        if i < 2:
            args = (aggA, aggB) + par + (gcn_W[i + 1],)
            if i > 0:
                args = args + (x,)
            x, hA, hB = _epi(i > 0)(*args)
        else:
            node_emb, plan_emb = _fin(aggA, aggB, *par, x,
                                      batch_idx.reshape(N, 1))
    return node_emb, plan_emb


# speedup vs baseline: 13.0755x; 13.0755x over previous
import functools

import jax
import jax.numpy as jnp
from jax import lax
from jax.experimental import pallas as pl
from jax.experimental.pallas import tpu as pltpu
from jax.experimental.pallas import tpu_sc as plsc

N = 50000
E = 800000
D_IN = 128
H = 64
HQ = 16
NB = 8

BLK = 2000
NP = 52000
NPS = 52096
GRID = NP // BLK
GRID_N = N // BLK
SINK = N

NC, NS = 2, 16
CHUNK = 128
E_PAD = 802816
AGG_STEPS = E_PAD // NS // CHUNK
STEPS_H = AGG_STEPS // 2
DEG_STEPS = E_PAD // (NC * NS) // CHUNK
ROWS_PT = NPS // NS

_f32 = jnp.float32


def _sc_mesh():
    return plsc.VectorSubcoreMesh(
        core_axis_name="c", subcore_axis_name="s", num_cores=NC, num_subcores=NS
    )


@functools.partial(
    pl.kernel,
    out_type=jax.ShapeDtypeStruct((NC * NPS, 16), _f32),
    mesh=_sc_mesh(),
    scratch_types=[
        pltpu.VMEM((DEG_STEPS, CHUNK), jnp.int32),
        pltpu.VMEM((CHUNK, 16), _f32),
        pltpu.VMEM_SHARED((NPS, 16), _f32),
    ],
    compiler_params=pltpu.CompilerParams(use_tc_tiling_on_sc=False),
)
def _sc_deg(dst2, ones_hbm, zeros_hbm, deg_out, dst_v, ones_v, acc):
    c = lax.axis_index("c")
    s = lax.axis_index("s")
    w = c * NS + s
    pltpu.sync_copy(dst2.at[pl.ds(w * DEG_STEPS, DEG_STEPS)], dst_v)
    pltpu.sync_copy(ones_hbm, ones_v)
    sl = pl.ds(s * ROWS_PT, ROWS_PT)
    pltpu.sync_copy(zeros_hbm, acc.at[sl])
    plsc.subcore_barrier()

    def st(j, carry):
        pltpu.sync_copy(ones_v, acc.at[dst_v.at[j]], add=True)
        return carry

    lax.fori_loop(0, DEG_STEPS, st, None)
    plsc.subcore_barrier()
    pltpu.sync_copy(acc.at[sl], deg_out.at[pl.ds(c * NPS + s * ROWS_PT, ROWS_PT)])


def _sc_agg_body(p, e, *refs):
    if e == 0:
        h2, comb, agg2, idx_v, rows2, acc, sems = refs
    else:
        h2, comb, prev, agg2, idx_v, rows2, acc, sems = refs
    c = lax.axis_index("c")
    s = lax.axis_index("s")
    sl = pl.ds(s * ROWS_PT, ROWS_PT)
    csl = pl.ds(c * NPS + s * ROWS_PT, ROWS_PT)
    q = c + 2 * p
    pltpu.sync_copy(
        comb.at[pl.ds(((q * NS + s) * 2 + e) * 2 * STEPS_H, 2 * STEPS_H)],
        idx_v,
    )
    if e == 0:
        pltpu.sync_copy(h2.at[pl.ds(q * NPS + s * ROWS_PT, ROWS_PT)], acc.at[sl])
    else:
        pltpu.sync_copy(prev.at[csl], acc.at[sl])
    plsc.subcore_barrier()
    pltpu.async_copy(h2.at[idx_v.at[0]], rows2.at[0], sems.at[0])

    def step(jj, carry2):
        cur = jj % 2
        nxt = 1 - cur

        @pl.when(jj + 1 < STEPS_H)
        def _():
            pltpu.async_copy(
                h2.at[idx_v.at[jj + 1]], rows2.at[nxt], sems.at[nxt]
            )

        pltpu.make_async_copy(
            h2.at[idx_v.at[jj]], rows2.at[cur], sems.at[cur]
        ).wait()
        pltpu.sync_copy(
            rows2.at[cur], acc.at[idx_v.at[STEPS_H + jj]], add=True
        )
        return carry2

    lax.fori_loop(0, STEPS_H, step, None)
    plsc.subcore_barrier()
    pltpu.sync_copy(acc.at[sl], agg2.at[csl])


def _make_sc_agg(p, e):
    return pl.kernel(
        functools.partial(_sc_agg_body, p, e),
        out_type=jax.ShapeDtypeStruct((NC * NPS, HQ), _f32),
        mesh=_sc_mesh(),
        scratch_types=[
            pltpu.VMEM((2 * STEPS_H, CHUNK), jnp.int32),
            pltpu.VMEM((2, CHUNK, HQ), _f32),
            pltpu.VMEM_SHARED((NPS, HQ), _f32),
            pltpu.SemaphoreType.DMA((2,)),
        ],
        compiler_params=pltpu.CompilerParams(use_tc_tiling_on_sc=False),
    )


_sc_aggs = {(p, e): _make_sc_agg(p, e) for p in (0, 1) for e in (0, 1)}


def _split4(h, h4_ref):
    for q in range(4):
        h4_ref[q] = h[:, q * HQ:(q + 1) * HQ]


def _tc0_body(nodes_ref, win_ref, bin_ref, w0_ref, d0_ref, d1_ref,
              h4_ref, dis_ref):
    x = jnp.dot(nodes_ref[...], win_ref[...], preferred_element_type=_f32)
    x = x + bin_ref[...]
    deg = d0_ref[...][:, :1] + d1_ref[...][:, :1] + 1.0
    dis = lax.rsqrt(deg)
    h = jnp.dot(x, w0_ref[...], preferred_element_type=_f32) * dis
    _split4(h, h4_ref)
    dis_ref[...] = dis


def _tc0(nodes, W_in, b_in, W0, d0, d1):
    return pl.pallas_call(
        _tc0_body,
        grid=(GRID,),
        in_specs=[
            pl.BlockSpec((BLK, D_IN), lambda i: (jnp.minimum(i, GRID_N - 1), 0)),
            pl.BlockSpec((D_IN, H), lambda i: (0, 0)),
            pl.BlockSpec((1, H), lambda i: (0, 0)),
            pl.BlockSpec((H, H), lambda i: (0, 0)),
            pl.BlockSpec((BLK, 16), lambda i: (i, 0)),
            pl.BlockSpec((BLK, 16), lambda i: (i, 0)),
        ],
        out_specs=[
            pl.BlockSpec((4, BLK, HQ), lambda i: (0, i, 0)),
            pl.BlockSpec((BLK, 1), lambda i: (i, 0)),
        ],
        out_shape=[
            jax.ShapeDtypeStruct((4, NPS, HQ), _f32),
            jax.ShapeDtypeStruct((NP, 1), _f32),
        ],
    )(nodes, W_in, b_in, W0, d0, d1)


def _post(a4_ref, dis, b, g, lb, xres):
    out = jnp.concatenate([a4_ref[q] for q in range(4)], axis=1) * dis + b
    mu = jnp.mean(out, axis=-1, keepdims=True)
    var = jnp.mean((out - mu) ** 2, axis=-1, keepdims=True)
    out = (out - mu) * lax.rsqrt(var + 1e-5) * g + lb
    out = jnp.maximum(out, 0.0)
    if xres is not None:
        out = out + xres
    return out


def _epi_body(with_res, *refs):
    if with_res:
        (a4_ref, dis_ref, b_ref, g_ref, lb_ref, wn_ref, xres_ref,
         x_ref, h4_ref) = refs
        xres = xres_ref[...]
    else:
        (a4_ref, dis_ref, b_ref, g_ref, lb_ref, wn_ref,
         x_ref, h4_ref) = refs
        xres = None
    dis = dis_ref[...]
    out = _post(a4_ref, dis, b_ref[...], g_ref[...], lb_ref[...], xres)
    x_ref[...] = out
    h = jnp.dot(out, wn_ref[...], preferred_element_type=_f32) * dis
    _split4(h, h4_ref)


def _epi(with_res):
    body = functools.partial(_epi_body, with_res)
    n_par = pl.BlockSpec((1, H), lambda i: (0, 0))
    in_specs = [
        pl.BlockSpec((4, BLK, HQ), lambda i: (0, i, 0)),
        pl.BlockSpec((BLK, 1), lambda i: (i, 0)),
        n_par, n_par, n_par,
        pl.BlockSpec((H, H), lambda i: (0, 0)),
    ]
    if with_res:
        in_specs.append(pl.BlockSpec((BLK, H), lambda i: (i, 0)))
    return pl.pallas_call(
        body,
        grid=(GRID,),
        in_specs=in_specs,
        out_specs=[
            pl.BlockSpec((BLK, H), lambda i: (i, 0)),
            pl.BlockSpec((4, BLK, HQ), lambda i: (0, i, 0)),
        ],
        out_shape=[
            jax.ShapeDtypeStruct((NP, H), _f32),
            jax.ShapeDtypeStruct((4, NPS, HQ), _f32),
        ],
    )


def _fin_body(a4_ref, dis_ref, b_ref, g_ref, lb_ref, xres_ref, bi_ref,
              node_ref, plan_ref):
    out = _post(a4_ref, dis_ref[...], b_ref[...], g_ref[...], lb_ref[...],
                xres_ref[...])
    node_ref[...] = out
    oh = (bi_ref[...] == lax.broadcasted_iota(jnp.int32, (BLK, NB), 1))
    part = lax.dot_general(oh.astype(_f32), out, (((0,), (0,)), ((), ())),
                           preferred_element_type=_f32)

    @pl.when(pl.program_id(0) == 0)
    def _():
        plan_ref[...] = jnp.zeros_like(plan_ref)

    plan_ref[...] += part


def _fin(a4, dis, b, g, lb, xres, bi):
    n_par = pl.BlockSpec((1, H), lambda i: (0, 0))
    return pl.pallas_call(
        _fin_body,
        grid=(GRID_N,),
        in_specs=[
            pl.BlockSpec((4, BLK, HQ), lambda i: (0, i, 0)),
            pl.BlockSpec((BLK, 1), lambda i: (i, 0)),
            n_par, n_par, n_par,
            pl.BlockSpec((BLK, H), lambda i: (i, 0)),
            pl.BlockSpec((BLK, 1), lambda i: (i, 0)),
        ],
        out_specs=[
            pl.BlockSpec((BLK, H), lambda i: (i, 0)),
            pl.BlockSpec((NB, H), lambda i: (0, 0)),
        ],
        out_shape=[
            jax.ShapeDtypeStruct((N, H), _f32),
            jax.ShapeDtypeStruct((NB, H), _f32),
        ],
    )(a4, dis, b, g, lb, xres, bi)


def kernel(nodes, edges, edge_types, batch_idx, W_in, b_in, emb, gcn_W, gcn_b,
           ln_g, ln_b):
    del edge_types, emb
    pad = jnp.full((E_PAD - E,), SINK, jnp.int32)
    src_flat = jnp.concatenate([edges[0], pad])
    dst_flat = jnp.concatenate([edges[1], pad])
    qoff = (jnp.arange(4, dtype=jnp.int32) * NPS)[:, None]
    srcq4 = (src_flat[None, :] + qoff).reshape(4, NS, 2, STEPS_H, CHUNK)
    dst4 = jnp.broadcast_to(
        dst_flat.reshape(1, NS, 2, STEPS_H, CHUNK), (4, NS, 2, STEPS_H, CHUNK)
    )
    comb = jnp.concatenate([srcq4, dst4], axis=3).reshape(
        4 * NS * 2 * 2 * STEPS_H, CHUNK
    )
    dst2 = dst_flat.reshape(NS * AGG_STEPS, CHUNK)
    ones16 = jnp.ones((CHUNK, 16), _f32)
    zeros16 = jnp.zeros((ROWS_PT, 16), _f32)

    degf = _sc_deg(dst2, ones16, zeros16)
    d0 = degf[:NPS]
    d1 = degf[NPS:]
    h4, dis = _tc0(nodes, W_in, b_in.reshape(1, H), gcn_W[0], d0, d1)
    x = None
    node_emb = plan_emb = None
    for i in range(3):
        h2 = h4.reshape(4 * NPS, HQ)
        a01 = _sc_aggs[(0, 1)](h2, comb, _sc_aggs[(0, 0)](h2, comb))
        a23 = _sc_aggs[(1, 1)](h2, comb, _sc_aggs[(1, 0)](h2, comb))
        a4 = jnp.concatenate([a01, a23]).reshape(4, NPS, HQ)
        par = (dis, gcn_b[i].reshape(1, H), ln_g[i].reshape(1, H),
               ln_b[i].reshape(1, H))
        if i < 2:
            args = (a4,) + par + (gcn_W[i + 1],)
            if i > 0:
                args = args + (x,)
            x, h4 = _epi(i > 0)(*args)
        else:
            node_emb, plan_emb = _fin(a4, *par, x, batch_idx.reshape(N, 1))
    return node_emb, plan_emb


# speedup vs baseline: 16.2085x; 1.2396x over previous
import functools

import jax
import jax.numpy as jnp
from jax import lax
from jax.experimental import pallas as pl
from jax.experimental.pallas import tpu as pltpu
from jax.experimental.pallas import tpu_sc as plsc

N = 50000
E = 800000
D_IN = 128
H = 64
HQ = 16
NB = 8

BLK = 2000
NP = 52000
NPS = 52096
GRID = NP // BLK
GRID_N = N // BLK
SINK = N

NC, NS = 2, 16
CHUNK = 128
E_PAD = 802816
AGG_STEPS = E_PAD // NS // CHUNK
STEPS_H = AGG_STEPS // 2
NBUF = 4
DEG_STEPS = E_PAD // (NC * NS) // CHUNK
ROWS_PT = NPS // NS

_f32 = jnp.float32


def _sc_mesh():
    return plsc.VectorSubcoreMesh(
        core_axis_name="c", subcore_axis_name="s", num_cores=NC, num_subcores=NS
    )


@functools.partial(
    pl.kernel,
    out_type=jax.ShapeDtypeStruct((NC * NPS, 16), _f32),
    mesh=_sc_mesh(),
    scratch_types=[
        pltpu.VMEM((DEG_STEPS, CHUNK), jnp.int32),
        pltpu.VMEM((CHUNK, 16), _f32),
        pltpu.VMEM_SHARED((NPS, 16), _f32),
    ],
    compiler_params=pltpu.CompilerParams(use_tc_tiling_on_sc=False),
)
def _sc_deg(dst2, ones_hbm, zeros_hbm, deg_out, dst_v, ones_v, acc):
    c = lax.axis_index("c")
    s = lax.axis_index("s")
    w = c * NS + s
    pltpu.sync_copy(dst2.at[pl.ds(w * DEG_STEPS, DEG_STEPS)], dst_v)
    pltpu.sync_copy(ones_hbm, ones_v)
    sl = pl.ds(s * ROWS_PT, ROWS_PT)
    pltpu.sync_copy(zeros_hbm, acc.at[sl])
    plsc.subcore_barrier()

    def st(j, carry):
        pltpu.sync_copy(ones_v, acc.at[dst_v.at[j]], add=True)
        return carry

    lax.fori_loop(0, DEG_STEPS, st, None)
    plsc.subcore_barrier()
    pltpu.sync_copy(acc.at[sl], deg_out.at[pl.ds(c * NPS + s * ROWS_PT, ROWS_PT)])


def _sc_agg_body(p, e, *refs):
    if e == 0:
        h2, comb, agg2, idx_v, rows2, acc, sems = refs
    else:
        h2, comb, prev, agg2, idx_v, rows2, acc, sems = refs
    c = lax.axis_index("c")
    s = lax.axis_index("s")
    sl = pl.ds(s * ROWS_PT, ROWS_PT)
    csl = pl.ds(c * NPS + s * ROWS_PT, ROWS_PT)
    q = c + 2 * p
    pltpu.sync_copy(
        comb.at[pl.ds(((q * NS + s) * 2 + e) * 2 * STEPS_H, 2 * STEPS_H)],
        idx_v,
    )
    if e == 0:
        pltpu.sync_copy(h2.at[pl.ds(q * NPS + s * ROWS_PT, ROWS_PT)], acc.at[sl])
    else:
        pltpu.sync_copy(prev.at[csl], acc.at[sl])
    plsc.subcore_barrier()
    for jp in range(NBUF - 1):
        pltpu.async_copy(h2.at[idx_v.at[jp]], rows2.at[jp], sems.at[jp])

    def step(jj, carry2):
        cur = jj % NBUF
        nxt = (jj + NBUF - 1) % NBUF

        @pl.when(jj + NBUF - 1 < STEPS_H)
        def _():
            pltpu.async_copy(
                h2.at[idx_v.at[jj + NBUF - 1]], rows2.at[nxt], sems.at[nxt]
            )

        pltpu.make_async_copy(
            h2.at[idx_v.at[jj]], rows2.at[cur], sems.at[cur]
        ).wait()
        pltpu.sync_copy(
            rows2.at[cur], acc.at[idx_v.at[STEPS_H + jj]], add=True
        )
        return carry2

    lax.fori_loop(0, STEPS_H, step, None)
    plsc.subcore_barrier()
    pltpu.sync_copy(acc.at[sl], agg2.at[csl])


def _make_sc_agg(p, e):
    return pl.kernel(
        functools.partial(_sc_agg_body, p, e),
        out_type=jax.ShapeDtypeStruct((NC * NPS, HQ), _f32),
        mesh=_sc_mesh(),
        scratch_types=[
            pltpu.VMEM((2 * STEPS_H, CHUNK), jnp.int32),
            pltpu.VMEM((NBUF, CHUNK, HQ), _f32),
            pltpu.VMEM_SHARED((NPS, HQ), _f32),
            pltpu.SemaphoreType.DMA((NBUF,)),
        ],
        compiler_params=pltpu.CompilerParams(use_tc_tiling_on_sc=False),
    )


_sc_aggs = {(p, e): _make_sc_agg(p, e) for p in (0, 1) for e in (0, 1)}


def _split4(h, h4_ref):
    for q in range(4):
        h4_ref[q] = h[:, q * HQ:(q + 1) * HQ]


def _tc0_body(nodes_ref, win_ref, bin_ref, w0_ref, d0_ref, d1_ref,
              h4_ref, dis_ref):
    x = jnp.dot(nodes_ref[...], win_ref[...], preferred_element_type=_f32)
    x = x + bin_ref[...]
    deg = d0_ref[...][:, :1] + d1_ref[...][:, :1] + 1.0
    dis = lax.rsqrt(deg)
    h = jnp.dot(x, w0_ref[...], preferred_element_type=_f32) * dis
    _split4(h, h4_ref)
    dis_ref[...] = dis


def _tc0(nodes, W_in, b_in, W0, d0, d1):
    return pl.pallas_call(
        _tc0_body,
        grid=(GRID,),
        in_specs=[
            pl.BlockSpec((BLK, D_IN), lambda i: (jnp.minimum(i, GRID_N - 1), 0)),
            pl.BlockSpec((D_IN, H), lambda i: (0, 0)),
            pl.BlockSpec((1, H), lambda i: (0, 0)),
            pl.BlockSpec((H, H), lambda i: (0, 0)),
            pl.BlockSpec((BLK, 16), lambda i: (i, 0)),
            pl.BlockSpec((BLK, 16), lambda i: (i, 0)),
        ],
        out_specs=[
            pl.BlockSpec((4, BLK, HQ), lambda i: (0, i, 0)),
            pl.BlockSpec((BLK, 1), lambda i: (i, 0)),
        ],
        out_shape=[
            jax.ShapeDtypeStruct((4, NPS, HQ), _f32),
            jax.ShapeDtypeStruct((NP, 1), _f32),
        ],
    )(nodes, W_in, b_in, W0, d0, d1)


def _post(a4_ref, dis, b, g, lb, xres):
    out = jnp.concatenate([a4_ref[q] for q in range(4)], axis=1) * dis + b
    mu = jnp.mean(out, axis=-1, keepdims=True)
    var = jnp.mean((out - mu) ** 2, axis=-1, keepdims=True)
    out = (out - mu) * lax.rsqrt(var + 1e-5) * g + lb
    out = jnp.maximum(out, 0.0)
    if xres is not None:
        out = out + xres
    return out


def _epi_body(with_res, *refs):
    if with_res:
        (a4_ref, dis_ref, b_ref, g_ref, lb_ref, wn_ref, xres_ref,
         x_ref, h4_ref) = refs
        xres = xres_ref[...]
    else:
        (a4_ref, dis_ref, b_ref, g_ref, lb_ref, wn_ref,
         x_ref, h4_ref) = refs
        xres = None
    dis = dis_ref[...]
    out = _post(a4_ref, dis, b_ref[...], g_ref[...], lb_ref[...], xres)
    x_ref[...] = out
    h = jnp.dot(out, wn_ref[...], preferred_element_type=_f32) * dis
    _split4(h, h4_ref)


def _epi(with_res):
    body = functools.partial(_epi_body, with_res)
    n_par = pl.BlockSpec((1, H), lambda i: (0, 0))
    in_specs = [
        pl.BlockSpec((4, BLK, HQ), lambda i: (0, i, 0)),
        pl.BlockSpec((BLK, 1), lambda i: (i, 0)),
        n_par, n_par, n_par,
        pl.BlockSpec((H, H), lambda i: (0, 0)),
    ]
    if with_res:
        in_specs.append(pl.BlockSpec((BLK, H), lambda i: (i, 0)))
    return pl.pallas_call(
        body,
        grid=(GRID,),
        in_specs=in_specs,
        out_specs=[
            pl.BlockSpec((BLK, H), lambda i: (i, 0)),
            pl.BlockSpec((4, BLK, HQ), lambda i: (0, i, 0)),
        ],
        out_shape=[
            jax.ShapeDtypeStruct((NP, H), _f32),
            jax.ShapeDtypeStruct((4, NPS, HQ), _f32),
        ],
    )


def _fin_body(a4_ref, dis_ref, b_ref, g_ref, lb_ref, xres_ref, bi_ref,
              node_ref, plan_ref):
    out = _post(a4_ref, dis_ref[...], b_ref[...], g_ref[...], lb_ref[...],
                xres_ref[...])
    node_ref[...] = out
    oh = (bi_ref[...] == lax.broadcasted_iota(jnp.int32, (BLK, NB), 1))
    part = lax.dot_general(oh.astype(_f32), out, (((0,), (0,)), ((), ())),
                           preferred_element_type=_f32)

    @pl.when(pl.program_id(0) == 0)
    def _():
        plan_ref[...] = jnp.zeros_like(plan_ref)

    plan_ref[...] += part


def _fin(a4, dis, b, g, lb, xres, bi):
    n_par = pl.BlockSpec((1, H), lambda i: (0, 0))
    return pl.pallas_call(
        _fin_body,
        grid=(GRID_N,),
        in_specs=[
            pl.BlockSpec((4, BLK, HQ), lambda i: (0, i, 0)),
            pl.BlockSpec((BLK, 1), lambda i: (i, 0)),
            n_par, n_par, n_par,
            pl.BlockSpec((BLK, H), lambda i: (i, 0)),
            pl.BlockSpec((BLK, 1), lambda i: (i, 0)),
        ],
        out_specs=[
            pl.BlockSpec((BLK, H), lambda i: (i, 0)),
            pl.BlockSpec((NB, H), lambda i: (0, 0)),
        ],
        out_shape=[
            jax.ShapeDtypeStruct((N, H), _f32),
            jax.ShapeDtypeStruct((NB, H), _f32),
        ],
    )(a4, dis, b, g, lb, xres, bi)


def kernel(nodes, edges, edge_types, batch_idx, W_in, b_in, emb, gcn_W, gcn_b,
           ln_g, ln_b):
    del edge_types, emb
    pad = jnp.full((E_PAD - E,), SINK, jnp.int32)
    src_flat = jnp.concatenate([edges[0], pad])
    dst_flat = jnp.concatenate([edges[1], pad])
    qoff = (jnp.arange(4, dtype=jnp.int32) * NPS)[:, None]
    srcq4 = (src_flat[None, :] + qoff).reshape(4, NS, 2, STEPS_H, CHUNK)
    dst4 = jnp.broadcast_to(
        dst_flat.reshape(1, NS, 2, STEPS_H, CHUNK), (4, NS, 2, STEPS_H, CHUNK)
    )
    comb = jnp.concatenate([srcq4, dst4], axis=3).reshape(
        4 * NS * 2 * 2 * STEPS_H, CHUNK
    )
    dst2 = dst_flat.reshape(NS * AGG_STEPS, CHUNK)
    ones16 = jnp.ones((CHUNK, 16), _f32)
    zeros16 = jnp.zeros((ROWS_PT, 16), _f32)

    degf = _sc_deg(dst2, ones16, zeros16)
    d0 = degf[:NPS]
    d1 = degf[NPS:]
    h4, dis = _tc0(nodes, W_in, b_in.reshape(1, H), gcn_W[0], d0, d1)
    x = None
    node_emb = plan_emb = None
    for i in range(3):
        h2 = h4.reshape(4 * NPS, HQ)
        a01 = _sc_aggs[(0, 1)](h2, comb, _sc_aggs[(0, 0)](h2, comb))
        a23 = _sc_aggs[(1, 1)](h2, comb, _sc_aggs[(1, 0)](h2, comb))
        a4 = jnp.concatenate([a01, a23]).reshape(4, NPS, HQ)
        par = (dis, gcn_b[i].reshape(1, H), ln_g[i].reshape(1, H),
               ln_b[i].reshape(1, H))
        if i < 2:
            args = (a4,) + par + (gcn_W[i + 1],)
            if i > 0:
                args = args + (x,)
            x, h4 = _epi(i > 0)(*args)
        else:
            node_emb, plan_emb = _fin(a4, *par, x, batch_idx.reshape(N, 1))
    return node_emb, plan_emb


# speedup vs baseline: 18.0374x; 1.1128x over previous
import functools

import jax
import jax.numpy as jnp
from jax import lax
from jax.experimental import pallas as pl
from jax.experimental.pallas import tpu as pltpu
from jax.experimental.pallas import tpu_sc as plsc

N = 50000
E = 800000
D_IN = 128
H = 64
HQ = 16
NB = 8

BLK = 2000
NP = 52000
NPS = 52096
GRID = NP // BLK
GRID_N = N // BLK
SINK = N

NC, NS = 2, 16
CHUNK = 128
E_PAD = 802816
AGG_STEPS = E_PAD // NS // CHUNK
STEPS_H = AGG_STEPS // 2
NBUF = 8
DEG_STEPS = E_PAD // (NC * NS) // CHUNK
ROWS_PT = NPS // NS

_f32 = jnp.float32


def _sc_mesh():
    return plsc.VectorSubcoreMesh(
        core_axis_name="c", subcore_axis_name="s", num_cores=NC, num_subcores=NS
    )


@functools.partial(
    pl.kernel,
    out_type=jax.ShapeDtypeStruct((NC * NPS, 16), _f32),
    mesh=_sc_mesh(),
    scratch_types=[
        pltpu.VMEM((DEG_STEPS, CHUNK), jnp.int32),
        pltpu.VMEM((CHUNK, 16), _f32),
        pltpu.VMEM_SHARED((NPS, 16), _f32),
    ],
    compiler_params=pltpu.CompilerParams(use_tc_tiling_on_sc=False),
)
def _sc_deg(dst2, ones_hbm, zeros_hbm, deg_out, dst_v, ones_v, acc):
    c = lax.axis_index("c")
    s = lax.axis_index("s")
    w = c * NS + s
    pltpu.sync_copy(dst2.at[pl.ds(w * DEG_STEPS, DEG_STEPS)], dst_v)
    pltpu.sync_copy(ones_hbm, ones_v)
    sl = pl.ds(s * ROWS_PT, ROWS_PT)
    pltpu.sync_copy(zeros_hbm, acc.at[sl])
    plsc.subcore_barrier()

    def st(j, carry):
        pltpu.sync_copy(ones_v, acc.at[dst_v.at[j]], add=True)
        return carry

    lax.fori_loop(0, DEG_STEPS, st, None)
    plsc.subcore_barrier()
    pltpu.sync_copy(acc.at[sl], deg_out.at[pl.ds(c * NPS + s * ROWS_PT, ROWS_PT)])


def _sc_agg_body(p, e, *refs):
    if e == 0:
        h2, comb, agg2, idx_v, rows2, acc, sems = refs
    else:
        h2, comb, prev, agg2, idx_v, rows2, acc, sems = refs
    c = lax.axis_index("c")
    s = lax.axis_index("s")
    sl = pl.ds(s * ROWS_PT, ROWS_PT)
    csl = pl.ds(c * NPS + s * ROWS_PT, ROWS_PT)
    q = c + 2 * p
    pltpu.sync_copy(
        comb.at[pl.ds(((q * NS + s) * 2 + e) * 2 * STEPS_H, 2 * STEPS_H)],
        idx_v,
    )
    if e == 0:
        pltpu.sync_copy(h2.at[pl.ds(q * NPS + s * ROWS_PT, ROWS_PT)], acc.at[sl])
    else:
        pltpu.sync_copy(prev.at[csl], acc.at[sl])
    plsc.subcore_barrier()
    for jp in range(NBUF - 1):
        pltpu.async_copy(h2.at[idx_v.at[jp]], rows2.at[jp], sems.at[jp])

    def step(jj, carry2):
        cur = jj % NBUF
        nxt = (jj + NBUF - 1) % NBUF

        @pl.when(jj + NBUF - 1 < STEPS_H)
        def _():
            pltpu.async_copy(
                h2.at[idx_v.at[jj + NBUF - 1]], rows2.at[nxt], sems.at[nxt]
            )

        pltpu.make_async_copy(
            h2.at[idx_v.at[jj]], rows2.at[cur], sems.at[cur]
        ).wait()
        pltpu.sync_copy(
            rows2.at[cur], acc.at[idx_v.at[STEPS_H + jj]], add=True
        )
        return carry2

    lax.fori_loop(0, STEPS_H, step, None)
    plsc.subcore_barrier()
    pltpu.sync_copy(acc.at[sl], agg2.at[csl])


def _make_sc_agg(p, e):
    return pl.kernel(
        functools.partial(_sc_agg_body, p, e),
        out_type=jax.ShapeDtypeStruct((NC * NPS, HQ), _f32),
        mesh=_sc_mesh(),
        scratch_types=[
            pltpu.VMEM((2 * STEPS_H, CHUNK), jnp.int32),
            pltpu.VMEM((NBUF, CHUNK, HQ), _f32),
            pltpu.VMEM_SHARED((NPS, HQ), _f32),
            pltpu.SemaphoreType.DMA((NBUF,)),
        ],
        compiler_params=pltpu.CompilerParams(use_tc_tiling_on_sc=False),
    )


_sc_aggs = {(p, e): _make_sc_agg(p, e) for p in (0, 1) for e in (0, 1)}


def _split4(h, h4_ref):
    for q in range(4):
        h4_ref[q] = h[:, q * HQ:(q + 1) * HQ]


def _tc0_body(nodes_ref, win_ref, bin_ref, w0_ref, d0_ref, d1_ref,
              h4_ref, dis_ref):
    x = jnp.dot(nodes_ref[...], win_ref[...], preferred_element_type=_f32)
    x = x + bin_ref[...]
    deg = d0_ref[...][:, :1] + d1_ref[...][:, :1] + 1.0
    dis = lax.rsqrt(deg)
    h = jnp.dot(x, w0_ref[...], preferred_element_type=_f32) * dis
    _split4(h, h4_ref)
    dis_ref[...] = dis


def _tc0(nodes, W_in, b_in, W0, d0, d1):
    return pl.pallas_call(
        _tc0_body,
        grid=(GRID,),
        in_specs=[
            pl.BlockSpec((BLK, D_IN), lambda i: (jnp.minimum(i, GRID_N - 1), 0)),
            pl.BlockSpec((D_IN, H), lambda i: (0, 0)),
            pl.BlockSpec((1, H), lambda i: (0, 0)),
            pl.BlockSpec((H, H), lambda i: (0, 0)),
            pl.BlockSpec((BLK, 16), lambda i: (i, 0)),
            pl.BlockSpec((BLK, 16), lambda i: (i, 0)),
        ],
        out_specs=[
            pl.BlockSpec((4, BLK, HQ), lambda i: (0, i, 0)),
            pl.BlockSpec((BLK, 1), lambda i: (i, 0)),
        ],
        out_shape=[
            jax.ShapeDtypeStruct((4, NPS, HQ), _f32),
            jax.ShapeDtypeStruct((NP, 1), _f32),
        ],
    )(nodes, W_in, b_in, W0, d0, d1)


def _post(a4_ref, dis, b, g, lb, xres):
    out = jnp.concatenate([a4_ref[q] for q in range(4)], axis=1) * dis + b
    mu = jnp.mean(out, axis=-1, keepdims=True)
    var = jnp.mean((out - mu) ** 2, axis=-1, keepdims=True)
    out = (out - mu) * lax.rsqrt(var + 1e-5) * g + lb
    out = jnp.maximum(out, 0.0)
    if xres is not None:
        out = out + xres
    return out


def _epi_body(with_res, *refs):
    if with_res:
        (a4_ref, dis_ref, b_ref, g_ref, lb_ref, wn_ref, xres_ref,
         x_ref, h4_ref) = refs
        xres = xres_ref[...]
    else:
        (a4_ref, dis_ref, b_ref, g_ref, lb_ref, wn_ref,
         x_ref, h4_ref) = refs
        xres = None
    dis = dis_ref[...]
    out = _post(a4_ref, dis, b_ref[...], g_ref[...], lb_ref[...], xres)
    x_ref[...] = out
    h = jnp.dot(out, wn_ref[...], preferred_element_type=_f32) * dis
    _split4(h, h4_ref)


def _epi(with_res):
    body = functools.partial(_epi_body, with_res)
    n_par = pl.BlockSpec((1, H), lambda i: (0, 0))
    in_specs = [
        pl.BlockSpec((4, BLK, HQ), lambda i: (0, i, 0)),
        pl.BlockSpec((BLK, 1), lambda i: (i, 0)),
        n_par, n_par, n_par,
        pl.BlockSpec((H, H), lambda i: (0, 0)),
    ]
    if with_res:
        in_specs.append(pl.BlockSpec((BLK, H), lambda i: (i, 0)))
    return pl.pallas_call(
        body,
        grid=(GRID,),
        in_specs=in_specs,
        out_specs=[
            pl.BlockSpec((BLK, H), lambda i: (i, 0)),
            pl.BlockSpec((4, BLK, HQ), lambda i: (0, i, 0)),
        ],
        out_shape=[
            jax.ShapeDtypeStruct((NP, H), _f32),
            jax.ShapeDtypeStruct((4, NPS, HQ), _f32),
        ],
    )


def _fin_body(a4_ref, dis_ref, b_ref, g_ref, lb_ref, xres_ref, bi_ref,
              node_ref, plan_ref):
    out = _post(a4_ref, dis_ref[...], b_ref[...], g_ref[...], lb_ref[...],
                xres_ref[...])
    node_ref[...] = out
    oh = (bi_ref[...] == lax.broadcasted_iota(jnp.int32, (BLK, NB), 1))
    part = lax.dot_general(oh.astype(_f32), out, (((0,), (0,)), ((), ())),
                           preferred_element_type=_f32)

    @pl.when(pl.program_id(0) == 0)
    def _():
        plan_ref[...] = jnp.zeros_like(plan_ref)

    plan_ref[...] += part


def _fin(a4, dis, b, g, lb, xres, bi):
    n_par = pl.BlockSpec((1, H), lambda i: (0, 0))
    return pl.pallas_call(
        _fin_body,
        grid=(GRID_N,),
        in_specs=[
            pl.BlockSpec((4, BLK, HQ), lambda i: (0, i, 0)),
            pl.BlockSpec((BLK, 1), lambda i: (i, 0)),
            n_par, n_par, n_par,
            pl.BlockSpec((BLK, H), lambda i: (i, 0)),
            pl.BlockSpec((BLK, 1), lambda i: (i, 0)),
        ],
        out_specs=[
            pl.BlockSpec((BLK, H), lambda i: (i, 0)),
            pl.BlockSpec((NB, H), lambda i: (0, 0)),
        ],
        out_shape=[
            jax.ShapeDtypeStruct((N, H), _f32),
            jax.ShapeDtypeStruct((NB, H), _f32),
        ],
    )(a4, dis, b, g, lb, xres, bi)


def kernel(nodes, edges, edge_types, batch_idx, W_in, b_in, emb, gcn_W, gcn_b,
           ln_g, ln_b):
    del edge_types, emb
    pad = jnp.full((E_PAD - E,), SINK, jnp.int32)
    src_flat = jnp.concatenate([edges[0], pad])
    dst_flat = jnp.concatenate([edges[1], pad])
    qoff = (jnp.arange(4, dtype=jnp.int32) * NPS)[:, None]
    srcq4 = (src_flat[None, :] + qoff).reshape(4, NS, 2, STEPS_H, CHUNK)
    dst4 = jnp.broadcast_to(
        dst_flat.reshape(1, NS, 2, STEPS_H, CHUNK), (4, NS, 2, STEPS_H, CHUNK)
    )
    comb = jnp.concatenate([srcq4, dst4], axis=3).reshape(
        4 * NS * 2 * 2 * STEPS_H, CHUNK
    )
    dst2 = dst_flat.reshape(NS * AGG_STEPS, CHUNK)
    ones16 = jnp.ones((CHUNK, 16), _f32)
    zeros16 = jnp.zeros((ROWS_PT, 16), _f32)

    degf = _sc_deg(dst2, ones16, zeros16)
    d0 = degf[:NPS]
    d1 = degf[NPS:]
    h4, dis = _tc0(nodes, W_in, b_in.reshape(1, H), gcn_W[0], d0, d1)
    x = None
    node_emb = plan_emb = None
    for i in range(3):
        h2 = h4.reshape(4 * NPS, HQ)
        a01 = _sc_aggs[(0, 1)](h2, comb, _sc_aggs[(0, 0)](h2, comb))
        a23 = _sc_aggs[(1, 1)](h2, comb, _sc_aggs[(1, 0)](h2, comb))
        a4 = jnp.concatenate([a01, a23]).reshape(4, NPS, HQ)
        par = (dis, gcn_b[i].reshape(1, H), ln_g[i].reshape(1, H),
               ln_b[i].reshape(1, H))
        if i < 2:
            args = (a4,) + par + (gcn_W[i + 1],)
            if i > 0:
                args = args + (x,)
            x, h4 = _epi(i > 0)(*args)
        else:
            node_emb, plan_emb = _fin(a4, *par, x, batch_idx.reshape(N, 1))
    return node_emb, plan_emb


# speedup vs baseline: 18.0934x; 1.0031x over previous
import functools

import jax
import jax.numpy as jnp
from jax import lax
from jax.experimental import pallas as pl
from jax.experimental.pallas import tpu as pltpu
from jax.experimental.pallas import tpu_sc as plsc

N = 50000
E = 800000
D_IN = 128
H = 64
HQ = 16
NB = 8

BLK = 2000
NP = 52000
NPS = 52096
GRID = NP // BLK
GRID_N = N // BLK
SINK = N

NC, NS = 2, 16
CHUNK = 128
E_PAD = 802816
AGG_STEPS = E_PAD // NS // CHUNK
STEPS_H = AGG_STEPS // 2
NBUF = 8
DEG_STEPS = E_PAD // (NC * NS) // CHUNK
ROWS_PT = NPS // NS

_f32 = jnp.float32


def _sc_mesh():
    return plsc.VectorSubcoreMesh(
        core_axis_name="c", subcore_axis_name="s", num_cores=NC, num_subcores=NS
    )


@functools.partial(
    pl.kernel,
    out_type=jax.ShapeDtypeStruct((NC * NPS, 16), _f32),
    mesh=_sc_mesh(),
    scratch_types=[
        pltpu.VMEM((DEG_STEPS, CHUNK), jnp.int32),
        pltpu.VMEM((CHUNK, 16), _f32),
        pltpu.VMEM_SHARED((NPS, 16), _f32),
        pltpu.SemaphoreType.DMA((NBUF,)),
    ],
    compiler_params=pltpu.CompilerParams(use_tc_tiling_on_sc=False),
)
def _sc_deg(dst2, ones_hbm, zeros_hbm, deg_out, dst_v, ones_v, acc, ssems):
    c = lax.axis_index("c")
    s = lax.axis_index("s")
    w = c * NS + s
    pltpu.sync_copy(dst2.at[pl.ds(w * DEG_STEPS, DEG_STEPS)], dst_v)
    pltpu.sync_copy(ones_hbm, ones_v)
    sl = pl.ds(s * ROWS_PT, ROWS_PT)
    pltpu.sync_copy(zeros_hbm, acc.at[sl])
    plsc.subcore_barrier()

    def st(j, carry):
        slot = j % NBUF

        @pl.when(j >= NBUF)
        def _():
            pltpu.make_async_copy(
                ones_v, acc.at[dst_v.at[j - NBUF]], ssems.at[slot]
            ).wait()

        pltpu.async_copy(ones_v, acc.at[dst_v.at[j]], ssems.at[slot], add=True)
        return carry

    lax.fori_loop(0, DEG_STEPS, st, None)
    for k in range(DEG_STEPS - NBUF, DEG_STEPS):
        pltpu.make_async_copy(
            ones_v, acc.at[dst_v.at[k]], ssems.at[k % NBUF]
        ).wait()
    plsc.subcore_barrier()
    pltpu.sync_copy(acc.at[sl], deg_out.at[pl.ds(c * NPS + s * ROWS_PT, ROWS_PT)])


def _sc_agg_body(p, e, *refs):
    if e == 0:
        h2, comb, agg2, idx_v, rows2, acc, gsems, ssems = refs
    else:
        h2, comb, prev, agg2, idx_v, rows2, acc, gsems, ssems = refs
    c = lax.axis_index("c")
    s = lax.axis_index("s")
    sl = pl.ds(s * ROWS_PT, ROWS_PT)
    csl = pl.ds(c * NPS + s * ROWS_PT, ROWS_PT)
    q = c + 2 * p
    pltpu.sync_copy(
        comb.at[pl.ds(((q * NS + s) * 2 + e) * 2 * STEPS_H, 2 * STEPS_H)],
        idx_v,
    )
    if e == 0:
        pltpu.sync_copy(h2.at[pl.ds(q * NPS + s * ROWS_PT, ROWS_PT)], acc.at[sl])
    else:
        pltpu.sync_copy(prev.at[csl], acc.at[sl])
    plsc.subcore_barrier()
    for jp in range(NBUF - 1):
        pltpu.async_copy(h2.at[idx_v.at[jp]], rows2.at[jp], gsems.at[jp])

    def step(jj, carry2):
        cur = jj % NBUF
        nxt = (jj + NBUF - 1) % NBUF

        @pl.when(jj + NBUF - 1 < STEPS_H)
        def _():
            @pl.when(jj >= 1)
            def _():
                pltpu.make_async_copy(
                    rows2.at[nxt],
                    acc.at[idx_v.at[STEPS_H + jj - 1]],
                    ssems.at[nxt],
                ).wait()

            pltpu.async_copy(
                h2.at[idx_v.at[jj + NBUF - 1]], rows2.at[nxt], gsems.at[nxt]
            )

        pltpu.make_async_copy(
            h2.at[idx_v.at[jj]], rows2.at[cur], gsems.at[cur]
        ).wait()
        pltpu.async_copy(
            rows2.at[cur], acc.at[idx_v.at[STEPS_H + jj]], ssems.at[cur],
            add=True,
        )
        return carry2

    lax.fori_loop(0, STEPS_H, step, None)
    for k in range(STEPS_H - NBUF, STEPS_H):
        pltpu.make_async_copy(
            rows2.at[k % NBUF], acc.at[idx_v.at[STEPS_H + k]],
            ssems.at[k % NBUF],
        ).wait()
    plsc.subcore_barrier()
    pltpu.sync_copy(acc.at[sl], agg2.at[csl])


def _make_sc_agg(p, e):
    return pl.kernel(
        functools.partial(_sc_agg_body, p, e),
        out_type=jax.ShapeDtypeStruct((NC * NPS, HQ), _f32),
        mesh=_sc_mesh(),
        scratch_types=[
            pltpu.VMEM((2 * STEPS_H, CHUNK), jnp.int32),
            pltpu.VMEM((NBUF, CHUNK, HQ), _f32),
            pltpu.VMEM_SHARED((NPS, HQ), _f32),
            pltpu.SemaphoreType.DMA((NBUF,)),
            pltpu.SemaphoreType.DMA((NBUF,)),
        ],
        compiler_params=pltpu.CompilerParams(use_tc_tiling_on_sc=False),
    )


_sc_aggs = {(p, e): _make_sc_agg(p, e) for p in (0, 1) for e in (0, 1)}


def _split4(h, h4_ref):
    for q in range(4):
        h4_ref[q] = h[:, q * HQ:(q + 1) * HQ]


def _tc0_body(nodes_ref, win_ref, bin_ref, w0_ref, d0_ref, d1_ref,
              h4_ref, dis_ref):
    x = jnp.dot(nodes_ref[...], win_ref[...], preferred_element_type=_f32)
    x = x + bin_ref[...]
    deg = d0_ref[...][:, :1] + d1_ref[...][:, :1] + 1.0
    dis = lax.rsqrt(deg)
    h = jnp.dot(x, w0_ref[...], preferred_element_type=_f32) * dis
    _split4(h, h4_ref)
    dis_ref[...] = dis


def _tc0(nodes, W_in, b_in, W0, d0, d1):
    return pl.pallas_call(
        _tc0_body,
        grid=(GRID,),
        in_specs=[
            pl.BlockSpec((BLK, D_IN), lambda i: (jnp.minimum(i, GRID_N - 1), 0)),
            pl.BlockSpec((D_IN, H), lambda i: (0, 0)),
            pl.BlockSpec((1, H), lambda i: (0, 0)),
            pl.BlockSpec((H, H), lambda i: (0, 0)),
            pl.BlockSpec((BLK, 16), lambda i: (i, 0)),
            pl.BlockSpec((BLK, 16), lambda i: (i, 0)),
        ],
        out_specs=[
            pl.BlockSpec((4, BLK, HQ), lambda i: (0, i, 0)),
            pl.BlockSpec((BLK, 1), lambda i: (i, 0)),
        ],
        out_shape=[
            jax.ShapeDtypeStruct((4, NPS, HQ), _f32),
            jax.ShapeDtypeStruct((NP, 1), _f32),
        ],
    )(nodes, W_in, b_in, W0, d0, d1)


def _post(a4_ref, dis, b, g, lb, xres):
    out = jnp.concatenate([a4_ref[q] for q in range(4)], axis=1) * dis + b
    mu = jnp.mean(out, axis=-1, keepdims=True)
    var = jnp.mean((out - mu) ** 2, axis=-1, keepdims=True)
    out = (out - mu) * lax.rsqrt(var + 1e-5) * g + lb
    out = jnp.maximum(out, 0.0)
    if xres is not None:
        out = out + xres
    return out


def _epi_body(with_res, *refs):
    if with_res:
        (a4_ref, dis_ref, b_ref, g_ref, lb_ref, wn_ref, xres_ref,
         x_ref, h4_ref) = refs
        xres = xres_ref[...]
    else:
        (a4_ref, dis_ref, b_ref, g_ref, lb_ref, wn_ref,
         x_ref, h4_ref) = refs
        xres = None
    dis = dis_ref[...]
    out = _post(a4_ref, dis, b_ref[...], g_ref[...], lb_ref[...], xres)
    x_ref[...] = out
    h = jnp.dot(out, wn_ref[...], preferred_element_type=_f32) * dis
    _split4(h, h4_ref)


def _epi(with_res):
    body = functools.partial(_epi_body, with_res)
    n_par = pl.BlockSpec((1, H), lambda i: (0, 0))
    in_specs = [
        pl.BlockSpec((4, BLK, HQ), lambda i: (0, i, 0)),
        pl.BlockSpec((BLK, 1), lambda i: (i, 0)),
        n_par, n_par, n_par,
        pl.BlockSpec((H, H), lambda i: (0, 0)),
    ]
    if with_res:
        in_specs.append(pl.BlockSpec((BLK, H), lambda i: (i, 0)))
    return pl.pallas_call(
        body,
        grid=(GRID,),
        in_specs=in_specs,
        out_specs=[
            pl.BlockSpec((BLK, H), lambda i: (i, 0)),
            pl.BlockSpec((4, BLK, HQ), lambda i: (0, i, 0)),
        ],
        out_shape=[
            jax.ShapeDtypeStruct((NP, H), _f32),
            jax.ShapeDtypeStruct((4, NPS, HQ), _f32),
        ],
    )


def _fin_body(a4_ref, dis_ref, b_ref, g_ref, lb_ref, xres_ref, bi_ref,
              node_ref, plan_ref):
    out = _post(a4_ref, dis_ref[...], b_ref[...], g_ref[...], lb_ref[...],
                xres_ref[...])
    node_ref[...] = out
    oh = (bi_ref[...] == lax.broadcasted_iota(jnp.int32, (BLK, NB), 1))
    part = lax.dot_general(oh.astype(_f32), out, (((0,), (0,)), ((), ())),
                           preferred_element_type=_f32)

    @pl.when(pl.program_id(0) == 0)
    def _():
        plan_ref[...] = jnp.zeros_like(plan_ref)

    plan_ref[...] += part


def _fin(a4, dis, b, g, lb, xres, bi):
    n_par = pl.BlockSpec((1, H), lambda i: (0, 0))
    return pl.pallas_call(
        _fin_body,
        grid=(GRID_N,),
        in_specs=[
            pl.BlockSpec((4, BLK, HQ), lambda i: (0, i, 0)),
            pl.BlockSpec((BLK, 1), lambda i: (i, 0)),
            n_par, n_par, n_par,
            pl.BlockSpec((BLK, H), lambda i: (i, 0)),
            pl.BlockSpec((BLK, 1), lambda i: (i, 0)),
        ],
        out_specs=[
            pl.BlockSpec((BLK, H), lambda i: (i, 0)),
            pl.BlockSpec((NB, H), lambda i: (0, 0)),
        ],
        out_shape=[
            jax.ShapeDtypeStruct((N, H), _f32),
            jax.ShapeDtypeStruct((NB, H), _f32),
        ],
    )(a4, dis, b, g, lb, xres, bi)


def kernel(nodes, edges, edge_types, batch_idx, W_in, b_in, emb, gcn_W, gcn_b,
           ln_g, ln_b):
    del edge_types, emb
    pad = jnp.full((E_PAD - E,), SINK, jnp.int32)
    src_flat = jnp.concatenate([edges[0], pad])
    dst_flat = jnp.concatenate([edges[1], pad])
    qoff = (jnp.arange(4, dtype=jnp.int32) * NPS)[:, None]
    srcq4 = (src_flat[None, :] + qoff).reshape(4, NS, 2, STEPS_H, CHUNK)
    dst4 = jnp.broadcast_to(
        dst_flat.reshape(1, NS, 2, STEPS_H, CHUNK), (4, NS, 2, STEPS_H, CHUNK)
    )
    comb = jnp.concatenate([srcq4, dst4], axis=3).reshape(
        4 * NS * 2 * 2 * STEPS_H, CHUNK
    )
    dst2 = dst_flat.reshape(NS * AGG_STEPS, CHUNK)
    ones16 = jnp.ones((CHUNK, 16), _f32)
    zeros16 = jnp.zeros((ROWS_PT, 16), _f32)

    degf = _sc_deg(dst2, ones16, zeros16)
    d0 = degf[:NPS]
    d1 = degf[NPS:]
    h4, dis = _tc0(nodes, W_in, b_in.reshape(1, H), gcn_W[0], d0, d1)
    x = None
    node_emb = plan_emb = None
    for i in range(3):
        h2 = h4.reshape(4 * NPS, HQ)
        a01 = _sc_aggs[(0, 1)](h2, comb, _sc_aggs[(0, 0)](h2, comb))
        a23 = _sc_aggs[(1, 1)](h2, comb, _sc_aggs[(1, 0)](h2, comb))
        a4 = jnp.concatenate([a01, a23]).reshape(4, NPS, HQ)
        par = (dis, gcn_b[i].reshape(1, H), ln_g[i].reshape(1, H),
               ln_b[i].reshape(1, H))
        if i < 2:
            args = (a4,) + par + (gcn_W[i + 1],)
            if i > 0:
                args = args + (x,)
            x, h4 = _epi(i > 0)(*args)
        else:
            node_emb, plan_emb = _fin(a4, *par, x, batch_idx.reshape(N, 1))
    return node_emb, plan_emb
